# baseline (device time: 97493 ns/iter reference)
import jax
import jax.numpy as jnp
from jax import lax
from jax.experimental import pallas as pl
from jax.experimental.pallas import tpu as pltpu

N_DEV = 4


def kernel(x, W1, W2):
    m_per, d = x.shape

    def body(x_ref, w1_ref, w2_ref, out_ref, xg_ref, acc_ref, stage_ref,
             ag_send, ag_recv, rs_send, rs_recv):
        my = lax.axis_index("i")
        left = (my - 1) % N_DEV
        right = (my + 1) % N_DEV

        barrier = pltpu.get_barrier_semaphore()
        for nbr in (left, right):
            pl.semaphore_signal(barrier, inc=1, device_id=(nbr,),
                                device_id_type=pl.DeviceIdType.MESH)
        pl.semaphore_wait(barrier, 2)

        xg_ref[0] = x_ref[...]

        for h in range(N_DEV - 1):
            rdma = pltpu.make_async_remote_copy(
                src_ref=xg_ref.at[h],
                dst_ref=xg_ref.at[h + 1],
                send_sem=ag_send.at[h],
                recv_sem=ag_recv.at[h],
                device_id=(right,),
                device_id_type=pl.DeviceIdType.MESH,
            )
            rdma.start()
            rdma.wait()

        for j in range(N_DEV):
            xb = xg_ref[(j + 1) % N_DEV]
            h1 = jnp.dot(xb, w1_ref[...], preferred_element_type=jnp.float32)
            h1 = h1 * jax.nn.sigmoid(h1)
            acc_ref[j] = jnp.dot(h1, w2_ref[...],
                                 preferred_element_type=jnp.float32)

        for s in range(N_DEV - 1):
            rdma = pltpu.make_async_remote_copy(
                src_ref=acc_ref.at[s],
                dst_ref=stage_ref.at[s],
                send_sem=rs_send.at[s],
                recv_sem=rs_recv.at[s],
                device_id=(right,),
                device_id_type=pl.DeviceIdType.MESH,
            )
            rdma.start()
            rdma.wait()
            if s < N_DEV - 2:
                acc_ref[s + 1] = acc_ref[s + 1] + stage_ref[s]

        out_ref[...] = acc_ref[N_DEV - 1] + stage_ref[N_DEV - 2]

    return pl.pallas_call(
        body,
        out_shape=jax.ShapeDtypeStruct((m_per, d), jnp.float32),
        in_specs=[pl.BlockSpec(memory_space=pltpu.VMEM)] * 3,
        out_specs=pl.BlockSpec(memory_space=pltpu.VMEM),
        scratch_shapes=[
            pltpu.VMEM((N_DEV, m_per, d), jnp.float32),
            pltpu.VMEM((N_DEV, m_per, d), jnp.float32),
            pltpu.VMEM((N_DEV - 1, m_per, d), jnp.float32),
            pltpu.SemaphoreType.DMA((N_DEV - 1,)),
            pltpu.SemaphoreType.DMA((N_DEV - 1,)),
            pltpu.SemaphoreType.DMA((N_DEV - 1,)),
            pltpu.SemaphoreType.DMA((N_DEV - 1,)),
        ],
        compiler_params=pltpu.CompilerParams(collective_id=0),
    )(x, W1, W2)


# device time: 52328 ns/iter; 1.8631x vs baseline; 1.8631x over previous
import jax
import jax.numpy as jnp
from jax import lax
from jax.experimental import pallas as pl
from jax.experimental.pallas import tpu as pltpu

N_DEV = 4


def kernel(x, W1, W2):
    m_per, d = x.shape
    mh = m_per // 2

    def body(x_ref, w1_ref, w2_ref, out_ref,
             xgA, accA, stgA, xgB, accB, stgB,
             agA_s, agA_r, rsA_s, rsA_r,
             agB_s, agB_r, rsB_s, rsB_r):
        my = lax.axis_index("i")
        left = (my - 1) % N_DEV
        right = (my + 1) % N_DEV

        barrier = pltpu.get_barrier_semaphore()
        for nbr in (left, right):
            pl.semaphore_signal(barrier, inc=1, device_id=(nbr,),
                                device_id_type=pl.DeviceIdType.MESH)
        pl.semaphore_wait(barrier, 2)

        def mk(src, dst, ssem, rsem, dev):
            return pltpu.make_async_remote_copy(
                src_ref=src, dst_ref=dst, send_sem=ssem, recv_sem=rsem,
                device_id=(dev,), device_id_type=pl.DeviceIdType.MESH)

        agA = [mk(xgA.at[h], xgA.at[h + 1], agA_s.at[h], agA_r.at[h], right)
               for h in range(N_DEV - 1)]
        agB = [mk(xgB.at[h], xgB.at[h + 1], agB_s.at[h], agB_r.at[h], left)
               for h in range(N_DEV - 1)]
        rsA = [mk(accA.at[s], stgA.at[s], rsA_s.at[s], rsA_r.at[s], right)
               for s in range(N_DEV - 1)]
        rsB = [mk(accB.at[s], stgB.at[s], rsB_s.at[s], rsB_r.at[s], left)
               for s in range(N_DEV - 1)]

        def partial(xb):
            h1 = jnp.dot(xb, w1_ref[...], preferred_element_type=jnp.float32)
            h1 = h1 * jax.nn.sigmoid(h1)
            return jnp.dot(h1, w2_ref[...], preferred_element_type=jnp.float32)

        xgA[0] = x_ref[0:mh, :]
        xgB[0] = x_ref[mh:, :]
        agA[0].start()
        agB[0].start()

        accA[3] = partial(xgA[0])
        accB[3] = partial(xgB[0])

        agA[0].wait_recv()
        agA[1].start()
        agB[0].wait_recv()
        agB[1].start()
        accA[0] = partial(xgA[1])
        accB[0] = partial(xgB[1])

        agA[1].wait_recv()
        agA[2].start()
        agB[1].wait_recv()
        agB[2].start()
        rsA[0].start()
        rsB[0].start()
        accA[1] = partial(xgA[2])
        accB[1] = partial(xgB[2])

        rsA[0].wait_recv()
        accA[1] = accA[1] + stgA[0]
        rsA[1].start()
        rsB[0].wait_recv()
        accB[1] = accB[1] + stgB[0]
        rsB[1].start()

        agA[2].wait_recv()
        accA[2] = partial(xgA[3])
        agB[2].wait_recv()
        accB[2] = partial(xgB[3])

        rsA[1].wait_recv()
        accA[2] = accA[2] + stgA[1]
        rsA[2].start()
        rsB[1].wait_recv()
        accB[2] = accB[2] + stgB[1]
        rsB[2].start()

        rsA[2].wait_recv()
        out_ref[0:mh, :] = accA[3] + stgA[2]
        rsB[2].wait_recv()
        out_ref[mh:, :] = accB[3] + stgB[2]

        for r in agA + agB + rsA + rsB:
            r.wait_send()

    half = pltpu.VMEM((N_DEV, mh, d), jnp.float32)
    stage = pltpu.VMEM((N_DEV - 1, mh, d), jnp.float32)
    sems = pltpu.SemaphoreType.DMA((N_DEV - 1,))
    return pl.pallas_call(
        body,
        out_shape=jax.ShapeDtypeStruct((m_per, d), jnp.float32),
        in_specs=[pl.BlockSpec(memory_space=pltpu.VMEM)] * 3,
        out_specs=pl.BlockSpec(memory_space=pltpu.VMEM),
        scratch_shapes=[
            half, half, stage,
            half, half, stage,
            sems, sems, sems, sems,
            sems, sems, sems, sems,
        ],
        compiler_params=pltpu.CompilerParams(collective_id=0),
    )(x, W1, W2)


# device time: 48965 ns/iter; 1.9911x vs baseline; 1.0687x over previous
import jax
import jax.numpy as jnp
from jax import lax
from jax.experimental import pallas as pl
from jax.experimental.pallas import tpu as pltpu

N_DEV = 4


def kernel(x, W1, W2):
    m_per, d = x.shape
    mh = m_per // 2

    def body(x_ref, w1_ref, w2_ref, out_ref,
             xgA, accA, stgA, xgB, accB, stgB,
             agA_s, agA_r, rsA_s, rsA_r,
             agB_s, agB_r, rsB_s, rsB_r):
        my = lax.axis_index("i")
        left = (my - 1) % N_DEV
        right = (my + 1) % N_DEV

        barrier = pltpu.get_barrier_semaphore()
        for nbr in (left, right):
            pl.semaphore_signal(barrier, inc=1, device_id=(nbr,),
                                device_id_type=pl.DeviceIdType.MESH)
        pl.semaphore_wait(barrier, 2)

        def mk(src, dst, ssem, rsem, dev):
            return pltpu.make_async_remote_copy(
                src_ref=src, dst_ref=dst, send_sem=ssem, recv_sem=rsem,
                device_id=(dev,), device_id_type=pl.DeviceIdType.MESH)

        agA = [mk(xgA.at[h], xgA.at[h + 1], agA_s.at[h], agA_r.at[h], right)
               for h in range(N_DEV - 1)]
        agB = [mk(xgB.at[h], xgB.at[h + 1], agB_s.at[h], agB_r.at[h], left)
               for h in range(N_DEV - 1)]
        rsA = [mk(accA.at[s], stgA.at[s], rsA_s.at[s], rsA_r.at[s], right)
               for s in range(N_DEV - 1)]
        rsB = [mk(accB.at[s], stgB.at[s], rsB_s.at[s], rsB_r.at[s], left)
               for s in range(N_DEV - 1)]

        def partial(xb):
            h1 = jnp.dot(xb, w1_ref[...], preferred_element_type=jnp.float32)
            h1 = h1 * jax.nn.sigmoid(h1)
            return jnp.dot(h1, w2_ref[...], preferred_element_type=jnp.float32)

        xgA[0] = x_ref[0:mh, :]
        xgB[0] = x_ref[mh:, :]
        agA[0].start()
        agB[0].start()

        accA[3] = partial(xgA[0])
        accB[3] = partial(xgB[0])

        agA[0].wait_recv()
        agA[1].start()
        agB[0].wait_recv()
        agB[1].start()
        accA[0] = partial(xgA[1])
        accB[0] = partial(xgB[1])
        rsA[0].start()
        rsB[0].start()

        agA[1].wait_recv()
        agA[2].start()
        agB[1].wait_recv()
        agB[2].start()
        accA[1] = partial(xgA[2])
        accB[1] = partial(xgB[2])

        rsA[0].wait_recv()
        accA[1] = accA[1] + stgA[0]
        rsA[1].start()
        rsB[0].wait_recv()
        accB[1] = accB[1] + stgB[0]
        rsB[1].start()

        agA[2].wait_recv()
        accA[2] = partial(xgA[3])
        agB[2].wait_recv()
        accB[2] = partial(xgB[3])

        rsA[1].wait_recv()
        accA[2] = accA[2] + stgA[1]
        rsA[2].start()
        rsB[1].wait_recv()
        accB[2] = accB[2] + stgB[1]
        rsB[2].start()

        rsA[2].wait_recv()
        out_ref[0:mh, :] = accA[3] + stgA[2]
        rsB[2].wait_recv()
        out_ref[mh:, :] = accB[3] + stgB[2]

        for r in agA + agB + rsA + rsB:
            r.wait_send()

    half = pltpu.VMEM((N_DEV, mh, d), jnp.float32)
    stage = pltpu.VMEM((N_DEV - 1, mh, d), jnp.float32)
    sems = pltpu.SemaphoreType.DMA((N_DEV - 1,))
    return pl.pallas_call(
        body,
        out_shape=jax.ShapeDtypeStruct((m_per, d), jnp.float32),
        in_specs=[pl.BlockSpec(memory_space=pltpu.VMEM)] * 3,
        out_specs=pl.BlockSpec(memory_space=pltpu.VMEM),
        scratch_shapes=[
            half, half, stage,
            half, half, stage,
            sems, sems, sems, sems,
            sems, sems, sems, sems,
        ],
        compiler_params=pltpu.CompilerParams(collective_id=0),
    )(x, W1, W2)
